# baseline (speedup 1.0000x reference)
import math

import numpy as np
import jax
import jax.numpy as jnp
from jax.experimental import pallas as pl
from jax.experimental.pallas import tpu as pltpu
from jax.experimental.shard_map import shard_map

_TWO_PI = 2.0 * math.pi
_INV_SQRT2 = 0.7071067811865476

_ERF_P = 0.3275911
_ERF_C = (0.254829592, -0.284496736, 1.421413741, -1.453152027, 1.061405429)


def _gelu(x):
    return 0.5 * x * (1.0 + jax.lax.erf(x * _INV_SQRT2))


def _ln_rows(y, n, gamma, beta):
    u = jnp.sum(y, axis=0, keepdims=True) * (1.0 / n)
    yc = y - u
    var = jnp.sum(yc * yc, axis=0, keepdims=True) * (1.0 / n)
    return yc * jax.lax.rsqrt(var + 1e-6) * gamma + beta


def _make_sparse_body(inv_w, inv_h, half):
    def body(coords_ref, labels_ref, gauss_ref, table_ref, out_ref):
        c = coords_ref[...] + 0.5
        cx = 2.0 * (c[:, 0:1] * inv_w) - 1.0
        cy = 2.0 * (c[:, 1:2] * inv_h) - 1.0
        g = gauss_ref[...]
        proj = _TWO_PI * (cx * g[0:1, :] + cy * g[1:2, :])

        lab = labels_ref[...]
        rows = jax.lax.broadcasted_iota(jnp.int32, (1, 5), 1)
        onehot = (lab + 1 == rows).astype(jnp.float32)
        add = jnp.dot(onehot, table_ref[...],
                      preferred_element_type=jnp.float32,
                      precision=jax.lax.Precision.HIGHEST)

        keep = (lab != -1).astype(jnp.float32)
        out_ref[:, :half] = jnp.sin(proj) * keep + add[:, :half]
        out_ref[:, half:] = jnp.cos(proj) * keep + add[:, half:]

    return body


def _sparse_embed(coords_flat, labels_flat, gauss, table, input_image_size):
    m = coords_flat.shape[0]
    d = table.shape[1]
    h_in, w_in = input_image_size
    body = _make_sparse_body(1.0 / w_in, 1.0 / h_in, d // 2)
    whole = lambda i: (0, 0)
    return pl.pallas_call(
        body,
        out_shape=jax.ShapeDtypeStruct((m, d), jnp.float32),
        grid=(1,),
        in_specs=[
            pl.BlockSpec((m, 2), whole),
            pl.BlockSpec((m, 1), whole),
            pl.BlockSpec((2, d // 2), whole),
            pl.BlockSpec((5, d), whole),
        ],
        out_specs=pl.BlockSpec((m, d), whole),
    )(coords_flat, labels_flat, gauss, table)


_NSTREAM = 4
_DEPTH = 2


def _make_dense_body(c1, c2, nsteps, d, npix):
    chunk = d // _NSTREAM

    def body(x_ref, w1_ref, b1_ref, g1_ref, e1_ref, a1_ref,
             w2_ref, b2_ref, g2_ref, e2_ref, w3_ref, b3_ref, out_ref):
        hp = jax.lax.Precision.HIGHEST
        x = x_ref[0]

        y1 = jnp.dot(w1_ref[...], x,
                     preferred_element_type=jnp.float32,
                     precision=hp) + b1_ref[...]
        a1 = a1_ref[...]
        u1 = jnp.dot(a1, y1,
                     preferred_element_type=jnp.float32, precision=hp)
        q1 = jnp.dot(a1, y1 * y1,
                     preferred_element_type=jnp.float32, precision=hp)
        v1 = q1 - u1 * u1
        yc1 = y1 - u1
        h1 = _gelu(yc1 * jax.lax.rsqrt(v1 + 1e-6) * g1_ref[...] + e1_ref[...])

        y2 = jnp.dot(w2_ref[...], h1,
                     preferred_element_type=jnp.float32,
                     precision=hp) + b2_ref[...]
        u2 = jnp.mean(y2, axis=0, keepdims=True)
        q2 = jnp.mean(y2 * y2, axis=0, keepdims=True)
        v2 = q2 - u2 * u2
        yc2 = y2 - u2
        h2 = _gelu(yc2 * jax.lax.rsqrt(v2 + 1e-6) * g2_ref[...] + e2_ref[...])

        out_ref[0] = jnp.dot(w3_ref[...], h2,
                             preferred_element_type=jnp.float32) + b3_ref[...]

    return body


def _dense_embed(xcols, w1s, b1t, ln1_g, ln1_b, w2_mat, conv2_b, ln2_g, ln2_b,
                 w3_mat, conv3_b):
    b, k16, npix = xcols.shape
    c1 = ln1_g.shape[0]
    c2 = w2_mat.shape[0]
    d = w3_mat.shape[0]
    body = _make_dense_body(c1, c2, b, d, npix)
    fixed = lambda i: (0, 0)

    g1t = jnp.tile(ln1_g, 4).reshape(4 * c1, 1)
    e1t = jnp.tile(ln1_b, 4).reshape(4 * c1, 1)
    eye4 = jnp.eye(4, dtype=jnp.float32)
    a1 = jnp.kron(eye4, jnp.full((c1, c1), 1.0 / c1, jnp.float32))

    return pl.pallas_call(
        body,
        out_shape=jax.ShapeDtypeStruct((b, d, npix), jnp.float32),
        grid=(b,),
        in_specs=[
            pl.BlockSpec((1, k16, npix), lambda i: (i, 0, 0)),
            pl.BlockSpec((k16, k16), fixed),
            pl.BlockSpec((k16, 1), fixed),
            pl.BlockSpec((4 * c1, 1), fixed),
            pl.BlockSpec((4 * c1, 1), fixed),
            pl.BlockSpec((k16, k16), fixed),
            pl.BlockSpec((c2, 4 * c1), fixed),
            pl.BlockSpec((c2, 1), fixed),
            pl.BlockSpec((c2, 1), fixed),
            pl.BlockSpec((c2, 1), fixed),
            pl.BlockSpec((d, c2), fixed),
            pl.BlockSpec((d, 1), fixed),
        ],
        out_specs=pl.BlockSpec((1, d, npix), lambda i: (i, 0, 0)),
        compiler_params=pltpu.CompilerParams(
            dimension_semantics=("arbitrary",)),
    )(xcols, w1s, b1t, g1t, e1t, a1,
      w2_mat, conv2_b.reshape(c2, 1),
      ln2_g.reshape(c2, 1), ln2_b.reshape(c2, 1),
      w3_mat, conv3_b.reshape(d, 1))


def _scatter_w1(w1_mat):
    c1 = w1_mat.shape[0]
    w = jnp.zeros((4 * c1, 16), jnp.float32)
    for a in range(2):
        for bb in range(2):
            q = a * 2 + bb
            for ky in range(2):
                for kx in range(2):
                    t = ky * 2 + kx
                    p = (2 * a + ky) * 4 + (2 * bb + kx)
                    w = w.at[q * c1:(q + 1) * c1, p].set(w1_mat[:, t])
    return w


def kernel(coords, labels, boxes, masks, pe_gaussian, emb_table, no_mask_embed,
           w1_mat, conv1_b, ln1_g, ln1_b, w2_mat, conv2_b, ln2_g, ln2_b,
           w3_mat, conv3_b):
    bs = coords.shape[0]
    d = emb_table.shape[1]
    input_image_size = (1024, 1024)

    def _sparse_path(crd, lab, box):
        nb = crd.shape[0]
        pts = crd.astype(jnp.float32)
        lbl = lab.astype(jnp.int32)
        bc = box.astype(jnp.float32).reshape(nb, 2, 2)
        bl = jnp.broadcast_to(jnp.array([[2, 3]], jnp.int32), (nb, 2))
        coords_all = jnp.concatenate([pts, bc], axis=1)
        labels_all = jnp.concatenate([lbl, bl], axis=1)
        n = coords_all.shape[1]
        return _sparse_embed(
            coords_all.reshape(nb * n, 2),
            labels_all.reshape(nb * n, 1),
            pe_gaussian, emb_table, input_image_size,
        ).reshape(nb, n, d)

    b, _, hm, wm = masks.shape
    he, we = hm // 4, wm // 4
    npix = he * we

    c1 = ln1_g.shape[0]
    w1s = _scatter_w1(w1_mat)
    b1t = jnp.tile(conv1_b, 4).reshape(4 * c1, 1)

    def _dense_path(m):
        xl = m.astype(jnp.float32).reshape(-1, he, 4, we, 4)
        xl = xl.transpose(0, 2, 4, 1, 3).reshape(-1, 16, npix)
        return _dense_embed(xl, w1s, b1t, ln1_g, ln1_b, w2_mat, conv2_b,
                            ln2_g, ln2_b, w3_mat, conv3_b)

    devs = jax.devices()
    P = jax.sharding.PartitionSpec
    if len(devs) >= 2 and b % 2 == 0 and bs == b and bs % 2 == 0:
        mesh = jax.sharding.Mesh(np.array(devs[:2]), ("b",))
        pb4 = P("b", None, None, None)
        msk = jax.lax.with_sharding_constraint(
            masks, jax.sharding.NamedSharding(mesh, pb4))
        sparse = shard_map(
            _sparse_path, mesh=mesh,
            in_specs=(P("b", None, None), P("b", None), P("b", None)),
            out_specs=P("b", None, None), check_rep=False,
        )(coords, labels, boxes)
        out = shard_map(
            _dense_path, mesh=mesh, in_specs=(pb4,),
            out_specs=P("b", None, None), check_rep=False,
        )(msk)
    else:
        sparse = _sparse_path(coords, labels, boxes)
        out = _dense_path(masks)
    dense = out.reshape(b, d, he, we)
    return sparse, dense


# speedup vs baseline: 1.1081x; 1.1081x over previous
import math

import numpy as np
import jax
import jax.numpy as jnp
from jax.experimental import pallas as pl
from jax.experimental.pallas import tpu as pltpu
from jax.experimental.shard_map import shard_map

_TWO_PI = 2.0 * math.pi
_INV_SQRT2 = 0.7071067811865476

_ERF_P = 0.3275911
_ERF_C = (0.254829592, -0.284496736, 1.421413741, -1.453152027, 1.061405429)


def _gelu(x):
    return 0.5 * x * (1.0 + jax.lax.erf(x * _INV_SQRT2))


def _ln_rows(y, n, gamma, beta):
    u = jnp.sum(y, axis=0, keepdims=True) * (1.0 / n)
    yc = y - u
    var = jnp.sum(yc * yc, axis=0, keepdims=True) * (1.0 / n)
    return yc * jax.lax.rsqrt(var + 1e-6) * gamma + beta


def _make_sparse_body(inv_w, inv_h, half):
    def body(coords_ref, labels_ref, gauss_ref, table_ref, out_ref):
        c = coords_ref[...] + 0.5
        cx = 2.0 * (c[:, 0:1] * inv_w) - 1.0
        cy = 2.0 * (c[:, 1:2] * inv_h) - 1.0
        g = gauss_ref[...]
        proj = _TWO_PI * (cx * g[0:1, :] + cy * g[1:2, :])

        lab = labels_ref[...]
        rows = jax.lax.broadcasted_iota(jnp.int32, (1, 5), 1)
        onehot = (lab + 1 == rows).astype(jnp.float32)
        add = jnp.dot(onehot, table_ref[...],
                      preferred_element_type=jnp.float32,
                      precision=jax.lax.Precision.HIGHEST)

        keep = (lab != -1).astype(jnp.float32)
        out_ref[:, :half] = jnp.sin(proj) * keep + add[:, :half]
        out_ref[:, half:] = jnp.cos(proj) * keep + add[:, half:]

    return body


def _sparse_embed(coords_flat, labels_flat, gauss, table, input_image_size):
    m = coords_flat.shape[0]
    d = table.shape[1]
    h_in, w_in = input_image_size
    body = _make_sparse_body(1.0 / w_in, 1.0 / h_in, d // 2)
    whole = lambda i: (0, 0)
    return pl.pallas_call(
        body,
        out_shape=jax.ShapeDtypeStruct((m, d), jnp.float32),
        grid=(1,),
        in_specs=[
            pl.BlockSpec((m, 2), whole),
            pl.BlockSpec((m, 1), whole),
            pl.BlockSpec((2, d // 2), whole),
            pl.BlockSpec((5, d), whole),
        ],
        out_specs=pl.BlockSpec((m, d), whole),
    )(coords_flat, labels_flat, gauss, table)


_NSTREAM = 4
_DEPTH = 2


def _make_dense_body(c1, c2, nsteps, d, npix):
    chunk = d // _NSTREAM

    def body(x_ref, w1_ref, b1_ref, g1_ref, e1_ref, a1_ref,
             w2_ref, b2_ref, g2_ref, e2_ref, w3_ref, b3_ref, out_ref,
             scratch_ref, sems_ref):
        hp = jax.lax.Precision.HIGHEST
        step = pl.program_id(0)
        slot = jax.lax.rem(step, _DEPTH)

        def _wait(sl, s):
            buf = scratch_ref.at[sl, pl.ds(s * chunk, chunk), :]
            pltpu.make_async_copy(buf, buf, sems_ref.at[sl, s]).wait()

        @pl.when(step >= _DEPTH)
        def _():
            for s in range(_NSTREAM):
                _wait(slot, s)

        x = x_ref[0]

        y1 = jnp.dot(w1_ref[...], x,
                     preferred_element_type=jnp.float32,
                     precision=hp) + b1_ref[...]
        a1 = a1_ref[...]
        u1 = jnp.dot(a1, y1,
                     preferred_element_type=jnp.float32, precision=hp)
        q1 = jnp.dot(a1, y1 * y1,
                     preferred_element_type=jnp.float32, precision=hp)
        v1 = q1 - u1 * u1
        yc1 = y1 - u1
        h1 = _gelu(yc1 * jax.lax.rsqrt(v1 + 1e-6) * g1_ref[...] + e1_ref[...])

        y2 = jnp.dot(w2_ref[...], h1,
                     preferred_element_type=jnp.float32,
                     precision=hp) + b2_ref[...]
        u2 = jnp.mean(y2, axis=0, keepdims=True)
        q2 = jnp.mean(y2 * y2, axis=0, keepdims=True)
        v2 = q2 - u2 * u2
        yc2 = y2 - u2
        h2 = _gelu(yc2 * jax.lax.rsqrt(v2 + 1e-6) * g2_ref[...] + e2_ref[...])

        scratch_ref[slot] = jnp.dot(w3_ref[...], h2,
                                    preferred_element_type=jnp.float32) \
            + b3_ref[...]

        for s in range(_NSTREAM):
            pltpu.make_async_copy(
                scratch_ref.at[slot, pl.ds(s * chunk, chunk), :],
                out_ref.at[step, pl.ds(s * chunk, chunk), :],
                sems_ref.at[slot, s]).start()

        @pl.when(step == nsteps - 1)
        def _():
            if nsteps >= 2:
                for s in range(_NSTREAM):
                    _wait(jax.lax.rem(step + 1, _DEPTH), s)
            for s in range(_NSTREAM):
                _wait(slot, s)

    return body


def _dense_embed(xcols, w1s, b1t, ln1_g, ln1_b, w2_mat, conv2_b, ln2_g, ln2_b,
                 w3_mat, conv3_b):
    b, k16, npix = xcols.shape
    c1 = ln1_g.shape[0]
    c2 = w2_mat.shape[0]
    d = w3_mat.shape[0]
    body = _make_dense_body(c1, c2, b, d, npix)
    fixed = lambda i: (0, 0)

    g1t = jnp.tile(ln1_g, 4).reshape(4 * c1, 1)
    e1t = jnp.tile(ln1_b, 4).reshape(4 * c1, 1)
    eye4 = jnp.eye(4, dtype=jnp.float32)
    a1 = jnp.kron(eye4, jnp.full((c1, c1), 1.0 / c1, jnp.float32))

    return pl.pallas_call(
        body,
        out_shape=jax.ShapeDtypeStruct((b, d, npix), jnp.float32),
        grid=(b,),
        in_specs=[
            pl.BlockSpec((1, k16, npix), lambda i: (i, 0, 0)),
            pl.BlockSpec((k16, k16), fixed),
            pl.BlockSpec((k16, 1), fixed),
            pl.BlockSpec((4 * c1, 1), fixed),
            pl.BlockSpec((4 * c1, 1), fixed),
            pl.BlockSpec((k16, k16), fixed),
            pl.BlockSpec((c2, 4 * c1), fixed),
            pl.BlockSpec((c2, 1), fixed),
            pl.BlockSpec((c2, 1), fixed),
            pl.BlockSpec((c2, 1), fixed),
            pl.BlockSpec((d, c2), fixed),
            pl.BlockSpec((d, 1), fixed),
        ],
        out_specs=pl.BlockSpec(memory_space=pl.ANY),
        scratch_shapes=[
            pltpu.VMEM((_DEPTH, d, npix), jnp.float32),
            pltpu.SemaphoreType.DMA((_DEPTH, _NSTREAM)),
        ],
        compiler_params=pltpu.CompilerParams(
            dimension_semantics=("arbitrary",)),
    )(xcols, w1s, b1t, g1t, e1t, a1,
      w2_mat, conv2_b.reshape(c2, 1),
      ln2_g.reshape(c2, 1), ln2_b.reshape(c2, 1),
      w3_mat, conv3_b.reshape(d, 1))


def _scatter_w1(w1_mat):
    c1 = w1_mat.shape[0]
    w = jnp.zeros((4 * c1, 16), jnp.float32)
    for a in range(2):
        for bb in range(2):
            q = a * 2 + bb
            for ky in range(2):
                for kx in range(2):
                    t = ky * 2 + kx
                    p = (2 * a + ky) * 4 + (2 * bb + kx)
                    w = w.at[q * c1:(q + 1) * c1, p].set(w1_mat[:, t])
    return w


def kernel(coords, labels, boxes, masks, pe_gaussian, emb_table, no_mask_embed,
           w1_mat, conv1_b, ln1_g, ln1_b, w2_mat, conv2_b, ln2_g, ln2_b,
           w3_mat, conv3_b):
    bs = coords.shape[0]
    d = emb_table.shape[1]
    input_image_size = (1024, 1024)

    def _sparse_path(crd, lab, box):
        nb = crd.shape[0]
        pts = crd.astype(jnp.float32)
        lbl = lab.astype(jnp.int32)
        bc = box.astype(jnp.float32).reshape(nb, 2, 2)
        bl = jnp.broadcast_to(jnp.array([[2, 3]], jnp.int32), (nb, 2))
        coords_all = jnp.concatenate([pts, bc], axis=1)
        labels_all = jnp.concatenate([lbl, bl], axis=1)
        n = coords_all.shape[1]
        return _sparse_embed(
            coords_all.reshape(nb * n, 2),
            labels_all.reshape(nb * n, 1),
            pe_gaussian, emb_table, input_image_size,
        ).reshape(nb, n, d)

    b, _, hm, wm = masks.shape
    he, we = hm // 4, wm // 4
    npix = he * we

    c1 = ln1_g.shape[0]
    w1s = _scatter_w1(w1_mat)
    b1t = jnp.tile(conv1_b, 4).reshape(4 * c1, 1)

    def _dense_path(m):
        xl = m.astype(jnp.float32).reshape(-1, he, 4, we, 4)
        xl = xl.transpose(0, 2, 4, 1, 3).reshape(-1, 16, npix)
        return _dense_embed(xl, w1s, b1t, ln1_g, ln1_b, w2_mat, conv2_b,
                            ln2_g, ln2_b, w3_mat, conv3_b)

    devs = jax.devices()
    P = jax.sharding.PartitionSpec
    if len(devs) >= 2 and b % 2 == 0 and bs == b and bs % 2 == 0:
        mesh = jax.sharding.Mesh(np.array(devs[:2]), ("b",))
        pb4 = P("b", None, None, None)
        msk = jax.lax.with_sharding_constraint(
            masks, jax.sharding.NamedSharding(mesh, pb4))
        sparse = shard_map(
            _sparse_path, mesh=mesh,
            in_specs=(P("b", None, None), P("b", None), P("b", None)),
            out_specs=P("b", None, None), check_rep=False,
        )(coords, labels, boxes)
        out = shard_map(
            _dense_path, mesh=mesh, in_specs=(pb4,),
            out_specs=P("b", None, None), check_rep=False,
        )(msk)
    else:
        sparse = _sparse_path(coords, labels, boxes)
        out = _dense_path(masks)
    dense = out.reshape(b, d, he, we)
    return sparse, dense
